# baseline (speedup 1.0000x reference)
import functools

import jax
import jax.numpy as jnp
from jax import lax
from jax.experimental import pallas as pl
from jax.experimental.pallas import tpu as pltpu
from jax.experimental.pallas import tpu_sc as plsc

NUM_CLASSES = 1024
DIM = 256
EMA = 0.99
N_ROWS = 16384

NC = 2
NS = 16
L = 16
NW = NC * NS
NG = 8
NQ = 4
Q_W = DIM // NQ
GROUP_ROWS = N_ROWS // NG
CHUNK = 128
N_CHUNKS = GROUP_ROWS // CHUNK
SUM_ROWS = NUM_CLASSES // 2
CNT_ROWS = NUM_CLASSES // 128
ACC_ROWS = SUM_ROWS + CNT_ROWS
ACC_W = 2 * Q_W


def _norm_body(x_ref, o_ref):
    x = x_ref[...]
    n2 = jnp.sum(x * x, axis=1, keepdims=True)
    inv = 1.0 / jnp.maximum(jnp.sqrt(n2), 1e-12)
    o_ref[...] = x * inv


def _normalize_rows(x):
    blk = 2048
    return pl.pallas_call(
        _norm_body,
        grid=(N_ROWS // blk,),
        in_specs=[pl.BlockSpec((blk, DIM), lambda i: (i, 0))],
        out_specs=pl.BlockSpec((blk, DIM), lambda i: (i, 0)),
        out_shape=jax.ShapeDtypeStruct((N_ROWS, DIM), jnp.float32),
    )(x)


def _sc_segment_sum(emb_norm, labels):
    mesh = plsc.VectorSubcoreMesh(
        core_axis_name="c", subcore_axis_name="s", num_cores=NC, num_subcores=NS
    )

    @functools.partial(
        pl.kernel,
        mesh=mesh,
        out_type=(
            jax.ShapeDtypeStruct((NW, SUM_ROWS, ACC_W), jnp.float32),
            jax.ShapeDtypeStruct((NG, CNT_ROWS, ACC_W), jnp.float32),
        ),
        scratch_types=[
            pltpu.VMEM((CHUNK, 2 * Q_W), jnp.float32),
            pltpu.VMEM((CHUNK, 2 * Q_W), jnp.float32),
            pltpu.VMEM((CHUNK,), jnp.int32),
            pltpu.VMEM((CHUNK,), jnp.int32),
            pltpu.VMEM((ACC_ROWS, ACC_W), jnp.float32),
            pltpu.SemaphoreType.DMA,
            pltpu.SemaphoreType.DMA,
        ],
        compiler_params=pltpu.CompilerParams(needs_layout_passes=False),
    )
    def k(emb_hbm, lab_hbm, out_sums, out_cnts, rowbuf0, rowbuf1,
          labbuf0, labbuf1, acc, sem0, sem1):
        cid = lax.axis_index("c")
        sid = lax.axis_index("s")
        wid = sid * NC + cid
        q = wid % NQ
        g = wid // NQ
        qoff = (q % 2) * Q_W

        zeros_v = jnp.zeros((L,), jnp.float32)
        ones_v = jnp.ones((L,), jnp.float32)
        col = lax.iota(jnp.int32, L)
        cnt_base = jnp.full((L,), SUM_ROWS, jnp.int32)
        is_q0 = jnp.full((L,), q, jnp.int32) == 0

        def z_rows(r, carry):
            for j in range(ACC_W // L):
                acc[r, pl.ds(j * L, L)] = zeros_v
            return carry

        lax.fori_loop(0, ACC_ROWS, z_rows, 0)

        rowbufs = (rowbuf0, rowbuf1)
        labbufs = (labbuf0, labbuf1)
        sems = (sem0, sem1)

        def start(kk):
            base = g * GROUP_ROWS + kk * CHUNK
            p = kk % 2
            rc = pltpu.async_copy(
                emb_hbm.at[pl.ds(base, CHUNK),
                           pl.ds((q // 2) * 2 * Q_W, 2 * Q_W)],
                rowbufs[p], sems[p],
            )
            lc = pltpu.async_copy(lab_hbm.at[pl.ds(base, CHUNK)],
                                  labbufs[p], sems[p])
            return rc, lc

        pend = start(0)
        for kk in range(N_CHUNKS):
            cur = kk % 2
            rc, lc = pend
            rc.wait()
            lc.wait()
            if kk + 1 < N_CHUNKS:
                pend = start(kk + 1)
            rowbuf = rowbufs[cur]
            labbuf = labbufs[cur]

            for j in range(CHUNK // L):
                lblv = labbuf[pl.ds(j * L, L)]
                plsc.addupdate_scatter(
                    acc, [cnt_base + (lblv >> 7), lblv & (ACC_W - 1)], ones_v,
                    mask=is_q0,
                )

            def row_body(i, carry):
                r0 = i * 8
                for u in range(8):
                    r = r0 + u
                    lblv = plsc.load_gather(
                        labbuf, [jnp.zeros((L,), jnp.int32) + r]
                    )
                    ridx = lblv >> 1
                    colr = col + ((lblv & 1) << 6)
                    for c in range(Q_W // L):
                        v = rowbuf[r, pl.ds(qoff + c * L, L)]
                        plsc.addupdate_scatter(acc, [ridx, colr + c * L], v)
                return carry

            lax.fori_loop(0, CHUNK // 8, row_body, 0)

        pltpu.sync_copy(acc.at[pl.ds(0, SUM_ROWS)], out_sums.at[wid])

        @pl.when(q == 0)
        def _():
            pltpu.sync_copy(acc.at[pl.ds(SUM_ROWS, CNT_ROWS)], out_cnts.at[g])

    return k(emb_norm, labels)


def _final_body(sums_ref, cnt_ref, proto_ref, init_ref, newp_ref, newi_ref):
    qs = []
    for qq in range(NQ):
        s = sums_ref[0, qq]
        for g in range(1, NG):
            s = s + sums_ref[g, qq]
        qs.append(s)
    sums = jnp.concatenate(qs, axis=1)
    cnt = cnt_ref[0]
    for g in range(1, NG):
        cnt = cnt + cnt_ref[g]
    mean = sums / jnp.maximum(cnt, 1.0)
    mn = jnp.sqrt(jnp.sum(mean * mean, axis=1, keepdims=True))
    m = mean / jnp.maximum(mn, 1e-12)
    proto = proto_ref[...]
    ema = EMA * proto + (1.0 - EMA) * m
    en = jnp.sqrt(jnp.sum(ema * ema, axis=1, keepdims=True))
    ema_n = ema / jnp.maximum(en, 1e-12)
    inited = init_ref[...] > 0
    has = cnt > 0.0
    upd = jnp.where(inited, ema_n, m)
    newp_ref[...] = jnp.where(has, upd, proto)
    newi_ref[...] = jnp.where(jnp.logical_or(inited, has), 1, 0)


def _finalize(sums_p, cnts_p, prototypes, init_i32):
    B = 256
    return pl.pallas_call(
        _final_body,
        grid=(NUM_CLASSES // B,),
        in_specs=[
            pl.BlockSpec((NG, NQ, B, Q_W), lambda i: (0, 0, i, 0)),
            pl.BlockSpec((NG, B, 1), lambda i: (0, i, 0)),
            pl.BlockSpec((B, DIM), lambda i: (i, 0)),
            pl.BlockSpec((B, 1), lambda i: (i, 0)),
        ],
        out_specs=[
            pl.BlockSpec((B, DIM), lambda i: (i, 0)),
            pl.BlockSpec((B, 1), lambda i: (i, 0)),
        ],
        out_shape=[
            jax.ShapeDtypeStruct((NUM_CLASSES, DIM), jnp.float32),
            jax.ShapeDtypeStruct((NUM_CLASSES, 1), jnp.int32),
        ],
    )(sums_p, cnts_p, prototypes, init_i32)


def kernel(embeddings, labels, prototypes, initialized):
    emb_n = _normalize_rows(embeddings)
    sums, cnts = _sc_segment_sum(emb_n, labels)
    sums_p = sums.reshape(NG, NQ, NUM_CLASSES, Q_W)
    cnts_p = cnts.reshape(NG, NUM_CLASSES, 1)
    init_i32 = initialized.astype(jnp.int32).reshape(NUM_CLASSES, 1)
    newp, newi = _finalize(sums_p, cnts_p, prototypes, init_i32)
    return newp, newi.reshape(NUM_CLASSES).astype(bool)


# speedup vs baseline: 1.0233x; 1.0233x over previous
import functools

import jax
import jax.numpy as jnp
from jax import lax
from jax.experimental import pallas as pl
from jax.experimental.pallas import tpu as pltpu
from jax.experimental.pallas import tpu_sc as plsc

NUM_CLASSES = 1024
DIM = 256
EMA = 0.99
N_ROWS = 16384

NC = 2
NS = 16
L = 16
NW = NC * NS
NG = 8
NQ = 4
Q_W = DIM // NQ
GROUP_ROWS = N_ROWS // NG
CHUNK = 128
N_CHUNKS = GROUP_ROWS // CHUNK
SUM_ROWS = NUM_CLASSES // 2
CNT_ROWS = NUM_CLASSES // 128
ACC_ROWS = SUM_ROWS + CNT_ROWS
ACC_W = 2 * Q_W


def _norm_body(x_ref, o_ref):
    x = x_ref[...]
    n2 = jnp.sum(x * x, axis=1, keepdims=True)
    inv = 1.0 / jnp.maximum(jnp.sqrt(n2), 1e-12)
    o_ref[...] = x * inv


def _normalize_rows(x):
    blk = 2048
    return pl.pallas_call(
        _norm_body,
        grid=(N_ROWS // blk,),
        in_specs=[pl.BlockSpec((blk, DIM), lambda i: (i, 0))],
        out_specs=pl.BlockSpec((blk, DIM), lambda i: (i, 0)),
        out_shape=jax.ShapeDtypeStruct((N_ROWS, DIM), jnp.float32),
    )(x)


def _sc_segment_sum(emb_norm, labels):
    mesh = plsc.VectorSubcoreMesh(
        core_axis_name="c", subcore_axis_name="s", num_cores=NC, num_subcores=NS
    )

    @functools.partial(
        pl.kernel,
        mesh=mesh,
        out_type=(
            jax.ShapeDtypeStruct((NW, SUM_ROWS, ACC_W), jnp.float32),
            jax.ShapeDtypeStruct((NW, CNT_ROWS, ACC_W), jnp.float32),
        ),
        scratch_types=[
            pltpu.VMEM((CHUNK, 2 * Q_W), jnp.float32),
            pltpu.VMEM((CHUNK, 2 * Q_W), jnp.float32),
            pltpu.VMEM((CHUNK,), jnp.int32),
            pltpu.VMEM((CHUNK,), jnp.int32),
            pltpu.VMEM((ACC_ROWS, ACC_W), jnp.float32),
            pltpu.SemaphoreType.DMA,
            pltpu.SemaphoreType.DMA,
        ],
        compiler_params=pltpu.CompilerParams(needs_layout_passes=False),
    )
    def k(emb_hbm, lab_hbm, out_sums, out_cnts, rowbuf0, rowbuf1,
          labbuf0, labbuf1, acc, sem0, sem1):
        cid = lax.axis_index("c")
        sid = lax.axis_index("s")
        wid = sid * NC + cid
        q = wid % NQ
        g = wid // NQ
        qoff = (q % 2) * Q_W

        zeros_v = jnp.zeros((L,), jnp.float32)
        ones_v = jnp.ones((L,), jnp.float32)
        col = lax.iota(jnp.int32, L)
        cnt_base = jnp.full((L,), SUM_ROWS, jnp.int32)
        is_q0 = jnp.full((L,), q, jnp.int32) == 0

        def z_rows(r, carry):
            for j in range(ACC_W // L):
                acc[r, pl.ds(j * L, L)] = zeros_v
            return carry

        lax.fori_loop(0, ACC_ROWS, z_rows, 0)

        rowbufs = (rowbuf0, rowbuf1)
        labbufs = (labbuf0, labbuf1)
        sems = (sem0, sem1)

        def start(kk):
            base = g * GROUP_ROWS + kk * CHUNK
            p = kk % 2
            rc = pltpu.async_copy(
                emb_hbm.at[pl.ds(base, CHUNK),
                           pl.ds((q // 2) * 2 * Q_W, 2 * Q_W)],
                rowbufs[p], sems[p],
            )
            lc = pltpu.async_copy(lab_hbm.at[pl.ds(base, CHUNK)],
                                  labbufs[p], sems[p])
            return rc, lc

        pend = start(0)
        for kk in range(N_CHUNKS):
            cur = kk % 2
            rc, lc = pend
            rc.wait()
            lc.wait()
            if kk + 1 < N_CHUNKS:
                pend = start(kk + 1)
            rowbuf = rowbufs[cur]
            labbuf = labbufs[cur]

            for j in range(CHUNK // L):
                lblv = labbuf[pl.ds(j * L, L)]
                plsc.addupdate_scatter(
                    acc, [cnt_base + (lblv >> 7), lblv & (ACC_W - 1)], ones_v,
                    mask=is_q0,
                )

            def row_body(i, carry):
                r0 = i * 8
                for u in range(8):
                    r = r0 + u
                    lblv = plsc.load_gather(
                        labbuf, [jnp.zeros((L,), jnp.int32) + r]
                    )
                    ridx = lblv >> 1
                    colr = col + ((lblv & 1) << 6)
                    for c in range(Q_W // L):
                        v = rowbuf[r, pl.ds(qoff + c * L, L)]
                        plsc.addupdate_scatter(acc, [ridx, colr + c * L], v)
                return carry

            lax.fori_loop(0, CHUNK // 8, row_body, 0)

        pltpu.sync_copy(acc.at[pl.ds(0, SUM_ROWS)], out_sums.at[wid])
        pltpu.sync_copy(acc.at[pl.ds(SUM_ROWS, CNT_ROWS)], out_cnts.at[wid])

    return k(emb_norm, labels)


def _final_body(sums_ref, cnt_ref, proto_ref, init_ref, newp_ref, newi_ref):
    qs = []
    for qq in range(NQ):
        s = sums_ref[0, qq]
        for g in range(1, NG):
            s = s + sums_ref[g, qq]
        qs.append(s)
    sums = jnp.concatenate(qs, axis=1)
    cnt = cnt_ref[0]
    for g in range(1, NG):
        cnt = cnt + cnt_ref[g]
    mean = sums / jnp.maximum(cnt, 1.0)
    mn = jnp.sqrt(jnp.sum(mean * mean, axis=1, keepdims=True))
    m = mean / jnp.maximum(mn, 1e-12)
    proto = proto_ref[...]
    ema = EMA * proto + (1.0 - EMA) * m
    en = jnp.sqrt(jnp.sum(ema * ema, axis=1, keepdims=True))
    ema_n = ema / jnp.maximum(en, 1e-12)
    inited = init_ref[...] > 0
    has = cnt > 0.0
    upd = jnp.where(inited, ema_n, m)
    newp_ref[...] = jnp.where(has, upd, proto)
    newi_ref[...] = jnp.where(jnp.logical_or(inited, has), 1, 0)


def _finalize(sums_p, cnts_p, prototypes, init_i32):
    B = 256
    return pl.pallas_call(
        _final_body,
        grid=(NUM_CLASSES // B,),
        in_specs=[
            pl.BlockSpec((NG, NQ, B, Q_W), lambda i: (0, 0, i, 0)),
            pl.BlockSpec((NG, B, 1), lambda i: (0, i, 0)),
            pl.BlockSpec((B, DIM), lambda i: (i, 0)),
            pl.BlockSpec((B, 1), lambda i: (i, 0)),
        ],
        out_specs=[
            pl.BlockSpec((B, DIM), lambda i: (i, 0)),
            pl.BlockSpec((B, 1), lambda i: (i, 0)),
        ],
        out_shape=[
            jax.ShapeDtypeStruct((NUM_CLASSES, DIM), jnp.float32),
            jax.ShapeDtypeStruct((NUM_CLASSES, 1), jnp.int32),
        ],
    )(sums_p, cnts_p, prototypes, init_i32)


def kernel(embeddings, labels, prototypes, initialized):
    emb_n = _normalize_rows(embeddings)
    sums, cnts = _sc_segment_sum(emb_n, labels)
    sums_p = sums.reshape(NG, NQ, NUM_CLASSES, Q_W)
    cnts_p = cnts.reshape(NW, NUM_CLASSES)[0::NQ].reshape(NG, NUM_CLASSES, 1)
    init_i32 = initialized.astype(jnp.int32).reshape(NUM_CLASSES, 1)
    newp, newi = _finalize(sums_p, cnts_p, prototypes, init_i32)
    return newp, newi.reshape(NUM_CLASSES).astype(bool)
